# Initial kernel scaffold; baseline (speedup 1.0000x reference)
import jax
import jax.numpy as jnp
from jax.experimental import pallas as pl


def kernel(upstream_flows, downstream_flows, feature, distance, edge_index, W11, b11, ln11_w, ln11_b, W12, b12, W21, b21, ln21_w, ln21_b, W22, b22, W3, b3, alpha, Wfc, Wattn):
    raise NotImplementedError("write your pallas kernel here")


# speedup vs baseline: 2.0165x; 2.0165x over previous
import functools

import jax
import jax.numpy as jnp
from jax import lax
from jax.experimental import pallas as pl
from jax.experimental.pallas import tpu as pltpu
from jax.experimental.pallas import tpu_sc as plsc

_N = 10000
_E = 160000
_B = 8
_T = 12
_H = 32
_EB = _E * _B

_NC = 2
_NS = 16
_NW = _NC * _NS
_CH = 128
_NCHUNKS = _E // _CH
_ROWS_PER_TILE = _N // _NS

_BLKR = 6400
_BLKE = 1000
_BLKN = 1000

_f32 = jnp.float32


def _sigmoid_poly(x):
    z = -x
    k = jnp.round(z * 1.4426950408889634)
    r = (z - k * 0.693359375) - k * (-2.12194440e-4)
    p = _f32(1.0 / 720.0)
    for c in (1.0 / 120.0, 1.0 / 24.0, 1.0 / 6.0, 0.5, 1.0, 1.0):
        p = p * r + _f32(c)
    return 1.0 / (1.0 + p * jnp.exp2(k))


def _vel_body(uf_ref, df_ref, w11t_ref, b11_ref, lnw1_ref, lnb1_ref,
              w12t_ref, w21t_ref, b21_ref, lnw2_ref, lnb2_ref, w22t_ref,
              sc_ref, v_ref):
    bf = jnp.bfloat16
    xu = jnp.dot(uf_ref[:].astype(bf), w11t_ref[:].astype(bf),
                 preferred_element_type=_f32) + b11_ref[:]
    mu_u = jnp.mean(xu, axis=1, keepdims=True)
    du = xu - mu_u
    var_u = jnp.mean(du * du, axis=1, keepdims=True)
    xu = du / jnp.sqrt(var_u + 1e-5) * lnw1_ref[:] + lnb1_ref[:]
    xu = jnp.maximum(xu, 0.0)
    pu = jnp.sum(xu.astype(bf).astype(_f32)
                 * w12t_ref[:].reshape(1, _H).astype(bf).astype(_f32),
                 axis=1, keepdims=True) + sc_ref[3]
    su = _sigmoid_poly(pu)
    xd = jnp.dot(df_ref[:].astype(bf), w21t_ref[:].astype(bf),
                 preferred_element_type=_f32) + b21_ref[:]
    mu_d = jnp.mean(xd, axis=1, keepdims=True)
    dd = xd - mu_d
    var_d = jnp.mean(dd * dd, axis=1, keepdims=True)
    xd = dd / jnp.sqrt(var_d + 1e-5) * lnw2_ref[:] + lnb2_ref[:]
    xd = jnp.maximum(xd, 0.0)
    pd = jnp.sum(xd.astype(bf).astype(_f32)
                 * w22t_ref[:].reshape(1, _H).astype(bf).astype(_f32),
                 axis=1, keepdims=True) + sc_ref[4]
    sd = _sigmoid_poly(pd)
    w30 = sc_ref[0].astype(bf).astype(_f32)
    w31 = sc_ref[1].astype(bf).astype(_f32)
    v_ref[:] = (w30 * su.astype(bf).astype(_f32)
                + w31 * sd.astype(bf).astype(_f32) + sc_ref[2])


def _run_vel(uf, df, w11t, b11r, lnw1, lnb1, w12t, w21t, b21r, lnw2, lnb2,
             w22t, scal):
    ga = _EB // _BLKR
    wspec = lambda shape: pl.BlockSpec(shape, lambda i: (0,) * len(shape))
    return pl.pallas_call(
        _vel_body,
        grid=(ga,),
        in_specs=[
            pl.BlockSpec((_BLKR, _T), lambda i: (i, 0)),
            pl.BlockSpec((_BLKR, _T), lambda i: (i, 0)),
            wspec((_T, _H)), wspec((1, _H)), wspec((1, _H)), wspec((1, _H)),
            wspec((_H, 1)),
            wspec((_T, _H)), wspec((1, _H)), wspec((1, _H)), wspec((1, _H)),
            wspec((_H, 1)),
            pl.BlockSpec(memory_space=pltpu.SMEM),
        ],
        out_specs=pl.BlockSpec((_BLKR, 1), lambda i: (i, 0)),
        out_shape=jax.ShapeDtypeStruct((_EB, 1), _f32),
    )(uf, df, w11t, b11r, lnw1, lnb1, w12t, w21t, b21r, lnw2, lnb2, w22t,
      scal)


def _node_body(f_ref, was_ref, wad_ref, featA_ref, adT_ref):
    f = f_ref[:]
    a_s = jnp.dot(f, was_ref[:], preferred_element_type=_f32,
                  precision=lax.Precision.HIGHEST)
    a_d = jnp.dot(f, wad_ref[:], preferred_element_type=_f32,
                  precision=lax.Precision.HIGHEST)
    featA_ref[:] = jnp.concatenate(
        [f, a_s, jnp.zeros((_BLKN, 8), _f32)], axis=1)
    adT_ref[:] = jnp.concatenate([a_d, jnp.zeros((_BLKN, 8), _f32)], axis=1)


def _run_node(f96, was96, wad96):
    gn = _N // _BLKN
    return pl.pallas_call(
        _node_body,
        grid=(gn,),
        in_specs=[
            pl.BlockSpec((_BLKN, _B * _T), lambda i: (i, 0)),
            pl.BlockSpec((_B * _T, _B), lambda i: (0, 0)),
            pl.BlockSpec((_B * _T, _B), lambda i: (0, 0)),
        ],
        out_specs=[
            pl.BlockSpec((_BLKN, 112), lambda i: (i, 0)),
            pl.BlockSpec((_BLKN, 16), lambda i: (i, 0)),
        ],
        out_shape=[
            jax.ShapeDtypeStruct((_N, 112), _f32),
            jax.ShapeDtypeStruct((_N, 16), _f32),
        ],
    )(f96, was96, wad96)


def _sc_mesh():
    return plsc.VectorSubcoreMesh(
        core_axis_name="c", subcore_axis_name="s", num_cores=_NC,
        num_subcores=_NS)


def _g1_body(featA_hbm, adT_hbm, src_hbm, dst_hbm, fsrc_hbm, addst_hbm,
             src_v, dst_v, rows_v, adrows_v, sem1, sem2):
    wid = lax.axis_index("s") * _NC + lax.axis_index("c")
    nch = jnp.where(wid < _NCHUNKS % _NW, _NCHUNKS // _NW + 1,
                    _NCHUNKS // _NW)

    def body(i, carry):
        base = (wid + i * _NW) * _CH
        pltpu.sync_copy(src_hbm.at[pl.ds(base, _CH)], src_v)
        pltpu.sync_copy(dst_hbm.at[pl.ds(base, _CH)], dst_v)
        cp1 = pltpu.async_copy(featA_hbm.at[src_v], rows_v, sem1)
        cp2 = pltpu.async_copy(adT_hbm.at[dst_v], adrows_v, sem2)
        cp1.wait()
        cp2.wait()
        pltpu.sync_copy(rows_v, fsrc_hbm.at[pl.ds(base, _CH)])
        pltpu.sync_copy(adrows_v, addst_hbm.at[pl.ds(base, _CH)])
        return carry

    lax.fori_loop(0, nch, body, 0)


@functools.cache
def _g1():
    return pl.kernel(
        _g1_body,
        out_type=(
            jax.ShapeDtypeStruct((_E, 112), _f32),
            jax.ShapeDtypeStruct((_E, 16), _f32),
        ),
        mesh=_sc_mesh(),
        scratch_types=[
            pltpu.VMEM((_CH,), jnp.int32),
            pltpu.VMEM((_CH,), jnp.int32),
            pltpu.VMEM((_CH, 112), _f32),
            pltpu.VMEM((_CH, 16), _f32),
            pltpu.SemaphoreType.DMA,
            pltpu.SemaphoreType.DMA,
        ],
        compiler_params=pltpu.CompilerParams(use_tc_tiling_on_sc=False),
    )


def _edge_body(fA_ref, addst_ref, v_ref, dist_ref, alpha_ref, rep_ref,
               c2_ref):
    fA = fA_ref[:]
    feat = fA[:, 0:96]
    asrc = fA[:, 96:104]
    a = asrc + addst_ref[:][:, 0:8]
    ee = jnp.exp(jnp.where(a > 0.0, a, 0.01 * a))
    v = v_ref[:]
    te = dist_ref[:] / (v + 1e-5)
    tq = jnp.round(te / 10.0)
    tq = jnp.where(tq < 0.0, 0.0, tq)
    tq = jnp.where(tq > float(_T), float(_T - 1), tq)
    f_coef = 1.0 / (1.0 + alpha_ref[:] * te)
    n = jnp.maximum(float(_T) - tq, 1.0)
    rep = rep_ref[:]
    f96 = jnp.dot(f_coef, rep, preferred_element_type=_f32,
                  precision=lax.Precision.HIGHEST)
    n96 = jnp.dot(n, rep, preferred_element_type=_f32,
                  precision=lax.Precision.HIGHEST)
    ti = lax.broadcasted_iota(jnp.int32, (_BLKE, 96), 1) % _T
    tf = ti.astype(_f32)
    mask = tf < n96
    ki = jnp.maximum((n96 - 1.0 - tf).astype(jnp.int32), 0)
    r = 1.0 - f96
    p = jnp.ones((_BLKE, 96), _f32)
    rc = r
    for bit in range(4):
        p = jnp.where(((ki >> bit) & 1) == 1, p * rc, p)
        rc = rc * rc
    w = jnp.where(mask, f96 * p, 0.0)
    sdot = lax.dot_general(w * feat, rep, (((1,), (1,)), ((), ())),
                           preferred_element_type=_f32,
                           precision=lax.Precision.HIGHEST)
    c2_ref[:] = jnp.concatenate([ee, ee * sdot], axis=1)


def _run_edge(fsrcA, addst, v8, dist2, alpha2, rep):
    ge = _E // _BLKE
    return pl.pallas_call(
        _edge_body,
        grid=(ge,),
        in_specs=[
            pl.BlockSpec((_BLKE, 112), lambda i: (i, 0)),
            pl.BlockSpec((_BLKE, 16), lambda i: (i, 0)),
            pl.BlockSpec((_BLKE, _B), lambda i: (i, 0)),
            pl.BlockSpec((_BLKE, 1), lambda i: (i, 0)),
            pl.BlockSpec((_BLKE, 1), lambda i: (i, 0)),
            pl.BlockSpec((_B, 96), lambda i: (0, 0)),
        ],
        out_specs=pl.BlockSpec((_BLKE, 16), lambda i: (i, 0)),
        out_shape=jax.ShapeDtypeStruct((_E, 16), _f32),
    )(fsrcA, addst, v8, dist2, alpha2, rep)


def _g2_body(c2_hbm, dst_hbm, zeros_hbm, parts_hbm, dst_v, vals_v, acc):
    cid = lax.axis_index("c")
    sid = lax.axis_index("s")
    wid = sid * _NC + cid
    row0 = sid * _ROWS_PER_TILE
    pltpu.sync_copy(zeros_hbm.at[pl.ds(row0, _ROWS_PER_TILE)],
                    acc.at[pl.ds(row0, _ROWS_PER_TILE)])
    plsc.subcore_barrier()
    nch = jnp.where(wid < _NCHUNKS % _NW, _NCHUNKS // _NW + 1,
                    _NCHUNKS // _NW)

    def body(i, carry):
        base = (wid + i * _NW) * _CH
        pltpu.sync_copy(dst_hbm.at[pl.ds(base, _CH)], dst_v)
        pltpu.sync_copy(c2_hbm.at[pl.ds(base, _CH)], vals_v)
        pltpu.sync_copy(vals_v, acc.at[dst_v], add=True)
        return carry

    lax.fori_loop(0, nch, body, 0)
    plsc.subcore_barrier()
    pltpu.sync_copy(acc.at[pl.ds(row0, _ROWS_PER_TILE)],
                    parts_hbm.at[cid, pl.ds(row0, _ROWS_PER_TILE)])


@functools.cache
def _g2():
    return pl.kernel(
        _g2_body,
        out_type=jax.ShapeDtypeStruct((_NC, _N, 16), _f32),
        mesh=_sc_mesh(),
        scratch_types=[
            pltpu.VMEM((_CH,), jnp.int32),
            pltpu.VMEM((_CH, 16), _f32),
            pltpu.VMEM_SHARED((_N, 16), _f32),
        ],
        compiler_params=pltpu.CompilerParams(use_tc_tiling_on_sc=False),
    )


def _fin_body(parts_ref, out_ref):
    p = parts_ref[:]
    s = p[0] + p[1]
    den = s[:, 0:8]
    num = s[:, 8:16]
    out_ref[:] = jnp.where(den > 0.0, num / den, 0.0)


def _run_fin(parts):
    gn = _N // _BLKN
    return pl.pallas_call(
        _fin_body,
        grid=(gn,),
        in_specs=[pl.BlockSpec((_NC, _BLKN, 16), lambda i: (0, i, 0))],
        out_specs=pl.BlockSpec((_BLKN, _B), lambda i: (i, 0)),
        out_shape=jax.ShapeDtypeStruct((_N, _B), _f32),
    )(parts)


def kernel(upstream_flows, downstream_flows, feature, distance, edge_index,
           W11, b11, ln11_w, ln11_b, W12, b12, W21, b21, ln21_w, ln21_b,
           W22, b22, W3, b3, alpha, Wfc, Wattn):
    uf = upstream_flows.reshape(_EB, _T)
    df = downstream_flows.reshape(_EB, _T)
    f96 = feature.reshape(_N, _B * _T)
    src = edge_index[0]
    dst = edge_index[1]
    dist2 = distance.reshape(_E, 1)

    w11t = W11.T
    w21t = W21.T
    w12t = W12.T
    w22t = W22.T
    b11r = b11.reshape(1, _H)
    b21r = b21.reshape(1, _H)
    lnw1 = ln11_w.reshape(1, _H)
    lnb1 = ln11_b.reshape(1, _H)
    lnw2 = ln21_w.reshape(1, _H)
    lnb2 = ln21_b.reshape(1, _H)
    scal = jnp.stack([W3[0, 0], W3[0, 1], b3[0], b12[0], b22[0]])
    vs_vec = Wfc.T @ Wattn[0, :_H]
    vd_vec = Wfc.T @ Wattn[0, _H:]
    eye8 = jnp.eye(_B, dtype=_f32)
    was96 = jnp.kron(eye8, vs_vec.reshape(_T, 1))
    wad96 = jnp.kron(eye8, vd_vec.reshape(_T, 1))
    rep = jnp.kron(eye8, jnp.ones((1, _T), _f32))

    v = _run_vel(uf, df, w11t, b11r, lnw1, lnb1, w12t, w21t, b21r, lnw2,
                 lnb2, w22t, scal)
    v8 = v.reshape(_E, _B)
    featA, adT = _run_node(f96, was96, wad96)
    fsrcA, addst = _g1()(featA, adT, src, dst)
    c2 = _run_edge(fsrcA, addst, v8, dist2, alpha, rep)
    parts = _g2()(c2, dst, jnp.zeros((_N, 16), _f32))
    return _run_fin(parts)
